# Initial kernel scaffold; baseline (speedup 1.0000x reference)
import jax
import jax.numpy as jnp
from jax.experimental import pallas as pl


def kernel(data_nodes, data_temporalEdges, data_spatialEdges, h_nodeRNN, c_nodeRNN, h_temporalEdgeRNN, c_temporalEdgeRNN, h_spatialEdgeRNN, c_spatialEdgeRNN, edge_index, W_enc_t, b_enc_t, Wih_t, Whh_t, bih_t, bhh_t, W_enc_s, b_enc_s, Wih_s, Whh_s, bih_s, bhh_s, W_enc_n, b_enc_n, W_ee, b_ee, Wih_n, Whh_n, bih_n, bhh_n, W_out, b_out):
    raise NotImplementedError("write your pallas kernel here")


# speedup vs baseline: 1.7440x; 1.7440x over previous
import functools

import jax
import jax.numpy as jnp
from jax import lax
from jax.experimental import pallas as pl
from jax.experimental.pallas import tpu as pltpu
from jax.experimental.pallas import tpu_sc as plsc

N = 10000
E = 160000
T = 4
RNN = 64
N_PAD = 10240
NTAB = 10368
HOLE = NTAB // 2
E_PAD = 163840
TILE_E = 1024
TILE_N = 512
NC = 2
NS = 16
PER_W = E_PAD // (NC * NS)
N_CH = PER_W // 128
ROWS_PT = NTAB // NS


def _lstm_update(xh, wcat, b, c):
    g = jnp.dot(xh, wcat, preferred_element_type=jnp.float32) + b
    i, f, gg, o = jnp.split(g, 4, axis=-1)
    c_new = jax.nn.sigmoid(f) * c + jax.nn.sigmoid(i) * jnp.tanh(gg)
    h_new = jax.nn.sigmoid(o) * jnp.tanh(c_new)
    return h_new, c_new


def _spatial_body(xs_ref, wenc_ref, benc_ref, wcat_ref, b_ref, out_ref):
    h = jnp.zeros((TILE_E, RNN), jnp.float32)
    c = jnp.zeros((TILE_E, RNN), jnp.float32)
    wenc = wenc_ref[...]
    wcat = wcat_ref[...]
    b = b_ref[...]
    benc = benc_ref[...]
    for t in range(T):
        x = xs_ref[t]
        enc = jnp.maximum(x[:, 0:1] * wenc[0:1, :] + x[:, 1:2] * wenc[1:2, :] + benc, 0.0)
        h, c = _lstm_update(jnp.concatenate([enc, h], axis=1), wcat, b, c)
        out_ref[t] = h


_spatial_call = pl.pallas_call(
    _spatial_body,
    grid=(E_PAD // TILE_E,),
    in_specs=[
        pl.BlockSpec((T, TILE_E, 2), lambda i: (0, i, 0)),
        pl.BlockSpec((2, RNN), lambda i: (0, 0)),
        pl.BlockSpec((1, RNN), lambda i: (0, 0)),
        pl.BlockSpec((2 * RNN, 4 * RNN), lambda i: (0, 0)),
        pl.BlockSpec((1, 4 * RNN), lambda i: (0, 0)),
    ],
    out_specs=pl.BlockSpec((T, TILE_E, RNN), lambda i: (0, i, 0)),
    out_shape=jax.ShapeDtypeStruct((T, E_PAD, RNN), jnp.float32),
)


N_GRP = PER_W // 64


def _agg_body(hs_hbm, gidx_hbm, zeros_hbm, out_hbm, idx_s, idx_d, buf, table):
    c = lax.axis_index("c")
    s = lax.axis_index("s")
    w = c * NS + s
    base = w * PER_W
    for t in range(T):
        pltpu.sync_copy(zeros_hbm.at[pl.ds(s * ROWS_PT, ROWS_PT)],
                        table.at[pl.ds(s * ROWS_PT, ROWS_PT)])
        plsc.subcore_barrier()

        def group(g, carry):
            pltpu.sync_copy(hs_hbm.at[t].at[pl.ds(base + g * 64, 64)],
                            buf.at[pl.ds(0, 64)])
            pltpu.sync_copy(gidx_hbm.at[w, g, 0], idx_s)
            pltpu.sync_copy(gidx_hbm.at[w, g, 1], idx_d)
            pltpu.sync_copy(buf, table.at[idx_s], add=True)
            pltpu.sync_copy(buf, table.at[idx_d], add=True)
            return carry

        lax.fori_loop(0, N_GRP, group, 0)
        plsc.subcore_barrier()
        pltpu.sync_copy(table.at[pl.ds(s * ROWS_PT, ROWS_PT)],
                        out_hbm.at[t, c].at[pl.ds(s * ROWS_PT, ROWS_PT)])
        plsc.subcore_barrier()


@functools.cache
def _get_agg_call():
    return pl.kernel(
        _agg_body,
        out_type=jax.ShapeDtypeStruct((T, NC, NTAB, RNN), jnp.float32),
        mesh=plsc.VectorSubcoreMesh(core_axis_name="c", subcore_axis_name="s"),
        scratch_types=[
            pltpu.VMEM((128,), jnp.int32),
            pltpu.VMEM((128,), jnp.int32),
            pltpu.VMEM((128, RNN), jnp.float32),
            pltpu.VMEM_SHARED((NTAB, RNN), jnp.float32),
        ],
    )


def _agg_run(hs_all, gidx, zeros_tab):
    return _get_agg_call()(hs_all, gidx, zeros_tab)


def _granule_idx(nodes):
    g = jnp.stack([2 * nodes, 2 * nodes + 1], axis=-1)
    return g.reshape(NC * NS, N_GRP, 128)


def _node_body(dte_ref, dn_ref, aggp_ref,
               wenc_t_ref, benc_t_ref, wcat_t_ref, b_t_ref,
               wenc_n_ref, benc_n_ref, wee_ref, bee_ref,
               wcat_n_ref, b_n_ref, wout_ref, bout_ref, out_ref):
    ht = jnp.zeros((TILE_N, RNN), jnp.float32)
    ct = jnp.zeros((TILE_N, RNN), jnp.float32)
    hn = jnp.zeros((TILE_N, RNN), jnp.float32)
    cn = jnp.zeros((TILE_N, RNN), jnp.float32)
    wenc_t = wenc_t_ref[...]
    wenc_n = wenc_n_ref[...]
    wee = wee_ref[...]
    wcat_t = wcat_t_ref[...]
    wcat_n = wcat_n_ref[...]
    for t in range(T):
        xe = dte_ref[t]
        enc_t = jnp.maximum(
            xe[:, 0:1] * wenc_t[0:1, :] + xe[:, 1:2] * wenc_t[1:2, :] + benc_t_ref[...], 0.0)
        ht, ct = _lstm_update(jnp.concatenate([enc_t, ht], axis=1), wcat_t, b_t_ref[...], ct)
        agg = aggp_ref[t, 0] + aggp_ref[t, 1]
        emb_e = jnp.maximum(
            jnp.dot(jnp.concatenate([ht, agg], axis=1), wee,
                    preferred_element_type=jnp.float32) + bee_ref[...], 0.0)
        enc_n = jnp.maximum(dn_ref[t] * wenc_n + benc_n_ref[...], 0.0)
        xh = jnp.concatenate([enc_n, emb_e, hn], axis=1)
        hn, cn = _lstm_update(xh, wcat_n, b_n_ref[...], cn)
        out_ref[t] = jnp.dot(hn, wout_ref[...], preferred_element_type=jnp.float32) + bout_ref[...]


_node_call = pl.pallas_call(
    _node_body,
    grid=(N_PAD // TILE_N,),
    in_specs=[
        pl.BlockSpec((T, TILE_N, 2), lambda i: (0, i, 0)),
        pl.BlockSpec((T, TILE_N, 1), lambda i: (0, i, 0)),
        pl.BlockSpec((T, NC, TILE_N, RNN), lambda i: (0, 0, i, 0)),
        pl.BlockSpec((2, RNN), lambda i: (0, 0)),
        pl.BlockSpec((1, RNN), lambda i: (0, 0)),
        pl.BlockSpec((2 * RNN, 4 * RNN), lambda i: (0, 0)),
        pl.BlockSpec((1, 4 * RNN), lambda i: (0, 0)),
        pl.BlockSpec((1, RNN), lambda i: (0, 0)),
        pl.BlockSpec((1, RNN), lambda i: (0, 0)),
        pl.BlockSpec((2 * RNN, RNN), lambda i: (0, 0)),
        pl.BlockSpec((1, RNN), lambda i: (0, 0)),
        pl.BlockSpec((3 * RNN, 4 * RNN), lambda i: (0, 0)),
        pl.BlockSpec((1, 4 * RNN), lambda i: (0, 0)),
        pl.BlockSpec((RNN, 8), lambda i: (0, 0)),
        pl.BlockSpec((1, 8), lambda i: (0, 0)),
    ],
    out_specs=pl.BlockSpec((T, TILE_N, 8), lambda i: (0, i, 0)),
    out_shape=jax.ShapeDtypeStruct((T, N_PAD, 8), jnp.float32),
)


def kernel(data_nodes, data_temporalEdges, data_spatialEdges, h_nodeRNN, c_nodeRNN,
           h_temporalEdgeRNN, c_temporalEdgeRNN, h_spatialEdgeRNN, c_spatialEdgeRNN,
           edge_index, W_enc_t, b_enc_t, Wih_t, Whh_t, bih_t, bhh_t,
           W_enc_s, b_enc_s, Wih_s, Whh_s, bih_s, bhh_s,
           W_enc_n, b_enc_n, W_ee, b_ee, Wih_n, Whh_n, bih_n, bhh_n,
           W_out, b_out):
    f32 = jnp.float32
    xs = jnp.pad(data_spatialEdges, ((0, 0), (0, E_PAD - E), (0, 0)))
    wcat_s = jnp.concatenate([Wih_s.T, Whh_s.T], axis=0)
    b_s = (bih_s + bhh_s).reshape(1, 4 * RNN)
    hs_all = _spatial_call(xs, W_enc_s.T, b_enc_s.reshape(1, RNN), wcat_s, b_s)

    pad_idx = jnp.full((E_PAD - E,), NTAB - 8, jnp.int32)
    def to_row(n):
        return n + 4 * (n >= HOLE).astype(jnp.int32)
    src = jnp.concatenate([to_row(edge_index[0]), pad_idx])
    dst = jnp.concatenate([to_row(edge_index[1]), pad_idx])
    gidx = jnp.stack([_granule_idx(src), _granule_idx(dst)], axis=2)
    zeros_tab = jnp.zeros((NTAB, RNN), f32)
    aggp = _agg_run(hs_all, gidx, zeros_tab)
    aggp = jnp.concatenate(
        [aggp[:, :, :HOLE], aggp[:, :, HOLE + 4:HOLE + 4 + (N_PAD - HOLE)]], axis=2)

    dte = jnp.pad(data_temporalEdges, ((0, 0), (0, N_PAD - N), (0, 0)))
    dn = jnp.pad(data_nodes, ((0, 0), (0, N_PAD - N), (0, 0)))
    wcat_t = jnp.concatenate([Wih_t.T, Whh_t.T], axis=0)
    b_t = (bih_t + bhh_t).reshape(1, 4 * RNN)
    wcat_n = jnp.concatenate([Wih_n.T, Whh_n.T], axis=0)
    b_n = (bih_n + bhh_n).reshape(1, 4 * RNN)
    wout = jnp.pad(W_out.T, ((0, 0), (0, 8 - 5)))
    bout = jnp.pad(b_out, ((0, 8 - 5))).reshape(1, 8)
    out = _node_call(dte, dn, aggp,
                     W_enc_t.T, b_enc_t.reshape(1, RNN), wcat_t, b_t,
                     W_enc_n.T, b_enc_n.reshape(1, RNN), W_ee.T, b_ee.reshape(1, RNN),
                     wcat_n, b_n, wout, bout)
    return out[:, :N, :5]


# speedup vs baseline: 2.3073x; 1.3230x over previous
import functools

import jax
import jax.numpy as jnp
from jax import lax
from jax.experimental import pallas as pl
from jax.experimental.pallas import tpu as pltpu
from jax.experimental.pallas import tpu_sc as plsc

N = 10000
E = 160000
T = 4
RNN = 64
N_PAD = 10240
NTAB = 10368
HOLE = NTAB // 2
E_PAD = 163840
TILE_E = 1024
TILE_N = 512
NC = 2
NS = 16
PER_W = E_PAD // (NC * NS)
N_CH = PER_W // 128
ROWS_PT = NTAB // NS


def _lstm_update(xh, wcat, b, c):
    g = jnp.dot(xh, wcat, preferred_element_type=jnp.float32) + b
    i, f, gg, o = jnp.split(g, 4, axis=-1)
    c_new = jax.nn.sigmoid(f) * c + jax.nn.sigmoid(i) * jnp.tanh(gg)
    h_new = jax.nn.sigmoid(o) * jnp.tanh(c_new)
    return h_new, c_new


def _spatial_body(xs_ref, wenc_ref, benc_ref, wcat_ref, b_ref, out_ref):
    h = jnp.zeros((TILE_E, RNN), jnp.float32)
    c = jnp.zeros((TILE_E, RNN), jnp.float32)
    wenc = wenc_ref[...]
    wcat = wcat_ref[...]
    b = b_ref[...]
    benc = benc_ref[...]
    for t in range(T):
        x = xs_ref[t]
        enc = jnp.maximum(x[:, 0:1] * wenc[0:1, :] + x[:, 1:2] * wenc[1:2, :] + benc, 0.0)
        h, c = _lstm_update(jnp.concatenate([enc, h], axis=1), wcat, b, c)
        out_ref[t] = h


_spatial_call = pl.pallas_call(
    _spatial_body,
    grid=(E_PAD // TILE_E,),
    in_specs=[
        pl.BlockSpec((T, TILE_E, 2), lambda i: (0, i, 0)),
        pl.BlockSpec((2, RNN), lambda i: (0, 0)),
        pl.BlockSpec((1, RNN), lambda i: (0, 0)),
        pl.BlockSpec((2 * RNN, 4 * RNN), lambda i: (0, 0)),
        pl.BlockSpec((1, 4 * RNN), lambda i: (0, 0)),
    ],
    out_specs=pl.BlockSpec((T, TILE_E, RNN), lambda i: (0, i, 0)),
    out_shape=jax.ShapeDtypeStruct((T, E_PAD, RNN), jnp.float32),
)


N_GRP = PER_W // 64


NBUF = 2


def _agg_body(hs_hbm, gidx_hbm, zeros_hbm, out_hbm, *scratch):
    bufs = scratch[0:NBUF]
    idx_ss = scratch[NBUF:2 * NBUF]
    idx_ds = scratch[2 * NBUF:3 * NBUF]
    load_sems = scratch[3 * NBUF:4 * NBUF]
    scat_sems = scratch[4 * NBUF:5 * NBUF]
    table = scratch[5 * NBUF]
    c = lax.axis_index("c")
    s = lax.axis_index("s")
    w = c * NS + s
    base = w * PER_W

    def issue_loads(t, g, k):
        pltpu.async_copy(hs_hbm.at[t].at[pl.ds(base + g * 64, 64)],
                         bufs[k].at[pl.ds(0, 64)], load_sems[k])
        pltpu.async_copy(gidx_hbm.at[w, g, 0], idx_ss[k], load_sems[k])
        pltpu.async_copy(gidx_hbm.at[w, g, 1], idx_ds[k], load_sems[k])

    def wait_loads(t, g, k):
        pltpu.make_async_copy(hs_hbm.at[t].at[pl.ds(base + g * 64, 64)],
                              bufs[k].at[pl.ds(0, 64)], load_sems[k]).wait()
        pltpu.make_async_copy(gidx_hbm.at[w, g, 0], idx_ss[k], load_sems[k]).wait()
        pltpu.make_async_copy(gidx_hbm.at[w, g, 1], idx_ds[k], load_sems[k]).wait()

    def do_scatters(k):
        pltpu.async_copy(bufs[k], table.at[idx_ss[k]], scat_sems[k], add=True)
        pltpu.make_async_copy(bufs[k], table.at[idx_ss[k]], scat_sems[k]).wait()
        pltpu.async_copy(bufs[k], table.at[idx_ds[k]], scat_sems[k], add=True)
        pltpu.make_async_copy(bufs[k], table.at[idx_ds[k]], scat_sems[k]).wait()

    for t in range(T):
        pltpu.sync_copy(zeros_hbm.at[pl.ds(s * ROWS_PT, ROWS_PT)],
                        table.at[pl.ds(s * ROWS_PT, ROWS_PT)])
        plsc.subcore_barrier()

        for k in range(NBUF - 1):
            issue_loads(t, k, k)

        def block(i, carry):
            g0 = i * NBUF
            for k in range(NBUF):
                g = g0 + k
                kr = (k + NBUF - 1) % NBUF
                @pl.when(g + NBUF - 1 < N_GRP)
                def _():
                    issue_loads(t, g + NBUF - 1, kr)
                wait_loads(t, g, k)
                do_scatters(k)
            return carry

        lax.fori_loop(0, N_GRP // NBUF, block, 0)
        plsc.subcore_barrier()
        pltpu.sync_copy(table.at[pl.ds(s * ROWS_PT, ROWS_PT)],
                        out_hbm.at[t, c].at[pl.ds(s * ROWS_PT, ROWS_PT)])
        plsc.subcore_barrier()


@functools.cache
def _get_agg_call():
    scratch = (
        [pltpu.VMEM((128, RNN), jnp.float32) for _ in range(NBUF)]
        + [pltpu.VMEM((128,), jnp.int32) for _ in range(NBUF)]
        + [pltpu.VMEM((128,), jnp.int32) for _ in range(NBUF)]
        + [pltpu.SemaphoreType.DMA for _ in range(NBUF)]
        + [pltpu.SemaphoreType.DMA for _ in range(NBUF)]
        + [pltpu.VMEM_SHARED((NTAB, RNN), jnp.float32)]
    )
    return pl.kernel(
        _agg_body,
        out_type=jax.ShapeDtypeStruct((T, NC, NTAB, RNN), jnp.float32),
        mesh=plsc.VectorSubcoreMesh(core_axis_name="c", subcore_axis_name="s"),
        scratch_types=scratch,
    )


def _agg_run(hs_all, gidx, zeros_tab):
    return _get_agg_call()(hs_all, gidx, zeros_tab)


def _granule_idx(nodes):
    g = jnp.stack([2 * nodes, 2 * nodes + 1], axis=-1)
    return g.reshape(NC * NS, N_GRP, 128)


def _node_body(dte_ref, dn_ref, aggp_ref,
               wenc_t_ref, benc_t_ref, wcat_t_ref, b_t_ref,
               wenc_n_ref, benc_n_ref, wee_ref, bee_ref,
               wcat_n_ref, b_n_ref, wout_ref, bout_ref, out_ref):
    ht = jnp.zeros((TILE_N, RNN), jnp.float32)
    ct = jnp.zeros((TILE_N, RNN), jnp.float32)
    hn = jnp.zeros((TILE_N, RNN), jnp.float32)
    cn = jnp.zeros((TILE_N, RNN), jnp.float32)
    wenc_t = wenc_t_ref[...]
    wenc_n = wenc_n_ref[...]
    wee = wee_ref[...]
    wcat_t = wcat_t_ref[...]
    wcat_n = wcat_n_ref[...]
    for t in range(T):
        xe = dte_ref[t]
        enc_t = jnp.maximum(
            xe[:, 0:1] * wenc_t[0:1, :] + xe[:, 1:2] * wenc_t[1:2, :] + benc_t_ref[...], 0.0)
        ht, ct = _lstm_update(jnp.concatenate([enc_t, ht], axis=1), wcat_t, b_t_ref[...], ct)
        agg = aggp_ref[t, 0] + aggp_ref[t, 1]
        emb_e = jnp.maximum(
            jnp.dot(jnp.concatenate([ht, agg], axis=1), wee,
                    preferred_element_type=jnp.float32) + bee_ref[...], 0.0)
        enc_n = jnp.maximum(dn_ref[t] * wenc_n + benc_n_ref[...], 0.0)
        xh = jnp.concatenate([enc_n, emb_e, hn], axis=1)
        hn, cn = _lstm_update(xh, wcat_n, b_n_ref[...], cn)
        out_ref[t] = jnp.dot(hn, wout_ref[...], preferred_element_type=jnp.float32) + bout_ref[...]


_node_call = pl.pallas_call(
    _node_body,
    grid=(N_PAD // TILE_N,),
    in_specs=[
        pl.BlockSpec((T, TILE_N, 2), lambda i: (0, i, 0)),
        pl.BlockSpec((T, TILE_N, 1), lambda i: (0, i, 0)),
        pl.BlockSpec((T, NC, TILE_N, RNN), lambda i: (0, 0, i, 0)),
        pl.BlockSpec((2, RNN), lambda i: (0, 0)),
        pl.BlockSpec((1, RNN), lambda i: (0, 0)),
        pl.BlockSpec((2 * RNN, 4 * RNN), lambda i: (0, 0)),
        pl.BlockSpec((1, 4 * RNN), lambda i: (0, 0)),
        pl.BlockSpec((1, RNN), lambda i: (0, 0)),
        pl.BlockSpec((1, RNN), lambda i: (0, 0)),
        pl.BlockSpec((2 * RNN, RNN), lambda i: (0, 0)),
        pl.BlockSpec((1, RNN), lambda i: (0, 0)),
        pl.BlockSpec((3 * RNN, 4 * RNN), lambda i: (0, 0)),
        pl.BlockSpec((1, 4 * RNN), lambda i: (0, 0)),
        pl.BlockSpec((RNN, 8), lambda i: (0, 0)),
        pl.BlockSpec((1, 8), lambda i: (0, 0)),
    ],
    out_specs=pl.BlockSpec((T, TILE_N, 8), lambda i: (0, i, 0)),
    out_shape=jax.ShapeDtypeStruct((T, N_PAD, 8), jnp.float32),
)


def kernel(data_nodes, data_temporalEdges, data_spatialEdges, h_nodeRNN, c_nodeRNN,
           h_temporalEdgeRNN, c_temporalEdgeRNN, h_spatialEdgeRNN, c_spatialEdgeRNN,
           edge_index, W_enc_t, b_enc_t, Wih_t, Whh_t, bih_t, bhh_t,
           W_enc_s, b_enc_s, Wih_s, Whh_s, bih_s, bhh_s,
           W_enc_n, b_enc_n, W_ee, b_ee, Wih_n, Whh_n, bih_n, bhh_n,
           W_out, b_out):
    f32 = jnp.float32
    xs = jnp.pad(data_spatialEdges, ((0, 0), (0, E_PAD - E), (0, 0)))
    wcat_s = jnp.concatenate([Wih_s.T, Whh_s.T], axis=0)
    b_s = (bih_s + bhh_s).reshape(1, 4 * RNN)
    hs_all = _spatial_call(xs, W_enc_s.T, b_enc_s.reshape(1, RNN), wcat_s, b_s)

    pad_idx = jnp.full((E_PAD - E,), NTAB - 8, jnp.int32)
    def to_row(n):
        return n + 4 * (n >= HOLE).astype(jnp.int32)
    src = jnp.concatenate([to_row(edge_index[0]), pad_idx])
    dst = jnp.concatenate([to_row(edge_index[1]), pad_idx])
    gidx = jnp.stack([_granule_idx(src), _granule_idx(dst)], axis=2)
    zeros_tab = jnp.zeros((NTAB, RNN), f32)
    aggp = _agg_run(hs_all, gidx, zeros_tab)
    aggp = jnp.concatenate(
        [aggp[:, :, :HOLE], aggp[:, :, HOLE + 4:HOLE + 4 + (N_PAD - HOLE)]], axis=2)

    dte = jnp.pad(data_temporalEdges, ((0, 0), (0, N_PAD - N), (0, 0)))
    dn = jnp.pad(data_nodes, ((0, 0), (0, N_PAD - N), (0, 0)))
    wcat_t = jnp.concatenate([Wih_t.T, Whh_t.T], axis=0)
    b_t = (bih_t + bhh_t).reshape(1, 4 * RNN)
    wcat_n = jnp.concatenate([Wih_n.T, Whh_n.T], axis=0)
    b_n = (bih_n + bhh_n).reshape(1, 4 * RNN)
    wout = jnp.pad(W_out.T, ((0, 0), (0, 8 - 5)))
    bout = jnp.pad(b_out, ((0, 8 - 5))).reshape(1, 8)
    out = _node_call(dte, dn, aggp,
                     W_enc_t.T, b_enc_t.reshape(1, RNN), wcat_t, b_t,
                     W_enc_n.T, b_enc_n.reshape(1, RNN), W_ee.T, b_ee.reshape(1, RNN),
                     wcat_n, b_n, wout, bout)
    return out[:, :N, :5]


# speedup vs baseline: 2.3157x; 1.0036x over previous
import functools

import jax
import jax.numpy as jnp
from jax import lax
from jax.experimental import pallas as pl
from jax.experimental.pallas import tpu as pltpu
from jax.experimental.pallas import tpu_sc as plsc

N = 10000
E = 160000
T = 4
RNN = 64
N_PAD = 10240
NTAB = 10368
HOLE = NTAB // 2
E_PAD = 163840
TILE_E = 1024
TILE_N = 512
NC = 2
NS = 16
PER_W = E_PAD // (NC * NS)
N_CH = PER_W // 128
ROWS_PT = NTAB // NS


def _lstm_update(xh, wcat, b, c):
    g = jnp.dot(xh, wcat, preferred_element_type=jnp.float32) + b
    i, f, gg, o = jnp.split(g, 4, axis=-1)
    c_new = jax.nn.sigmoid(f) * c + jax.nn.sigmoid(i) * jnp.tanh(gg)
    h_new = jax.nn.sigmoid(o) * jnp.tanh(c_new)
    return h_new, c_new


def _spatial_body(xs_ref, wenc_ref, benc_ref, wcat_ref, b_ref, out_ref):
    h = jnp.zeros((TILE_E, RNN), jnp.float32)
    c = jnp.zeros((TILE_E, RNN), jnp.float32)
    wenc = wenc_ref[...]
    wcat = wcat_ref[...]
    b = b_ref[...]
    benc = benc_ref[...]
    for t in range(T):
        x = xs_ref[t]
        enc = jnp.maximum(x[:, 0:1] * wenc[0:1, :] + x[:, 1:2] * wenc[1:2, :] + benc, 0.0)
        h, c = _lstm_update(jnp.concatenate([enc, h], axis=1), wcat, b, c)
        out_ref[t] = h


_spatial_call = pl.pallas_call(
    _spatial_body,
    grid=(E_PAD // TILE_E,),
    in_specs=[
        pl.BlockSpec((T, TILE_E, 2), lambda i: (0, i, 0)),
        pl.BlockSpec((2, RNN), lambda i: (0, 0)),
        pl.BlockSpec((1, RNN), lambda i: (0, 0)),
        pl.BlockSpec((2 * RNN, 4 * RNN), lambda i: (0, 0)),
        pl.BlockSpec((1, 4 * RNN), lambda i: (0, 0)),
    ],
    out_specs=pl.BlockSpec((T, TILE_E, RNN), lambda i: (0, i, 0)),
    out_shape=jax.ShapeDtypeStruct((T, E_PAD, RNN), jnp.float32),
)


N_GRP = PER_W // 64


NBUF = 2


def _agg_body(hs_hbm, gidx_hbm, zeros_hbm, out_hbm, *scratch):
    bufs = scratch[0:NBUF]
    idx_ss = scratch[NBUF:2 * NBUF]
    idx_ds = scratch[2 * NBUF:3 * NBUF]
    load_sems = scratch[3 * NBUF:4 * NBUF]
    scat_sems = scratch[4 * NBUF:5 * NBUF]
    table = scratch[5 * NBUF]
    c = lax.axis_index("c")
    s = lax.axis_index("s")
    w = c * NS + s
    base = w * PER_W

    def issue_loads(t, g, k):
        pltpu.async_copy(hs_hbm.at[t].at[pl.ds(base + g * 64, 64)],
                         bufs[k].at[pl.ds(0, 64)], load_sems[k])
        pltpu.async_copy(gidx_hbm.at[w, g, 0], idx_ss[k], load_sems[k])
        pltpu.async_copy(gidx_hbm.at[w, g, 1], idx_ds[k], load_sems[k])

    def wait_loads(t, g, k):
        pltpu.make_async_copy(hs_hbm.at[t].at[pl.ds(base + g * 64, 64)],
                              bufs[k].at[pl.ds(0, 64)], load_sems[k]).wait()
        pltpu.make_async_copy(gidx_hbm.at[w, g, 0], idx_ss[k], load_sems[k]).wait()
        pltpu.make_async_copy(gidx_hbm.at[w, g, 1], idx_ds[k], load_sems[k]).wait()

    def do_scatters(k):
        pltpu.async_copy(bufs[k], table.at[idx_ss[k]], scat_sems[k], add=True)
        pltpu.async_copy(bufs[k], table.at[idx_ds[k]], scat_sems[k], add=True)
        pltpu.make_async_copy(bufs[k], table.at[idx_ss[k]], scat_sems[k]).wait()
        pltpu.make_async_copy(bufs[k], table.at[idx_ds[k]], scat_sems[k]).wait()

    for t in range(T):
        pltpu.sync_copy(zeros_hbm.at[pl.ds(s * ROWS_PT, ROWS_PT)],
                        table.at[pl.ds(s * ROWS_PT, ROWS_PT)])
        plsc.subcore_barrier()

        for k in range(NBUF - 1):
            issue_loads(t, k, k)

        def block(i, carry):
            g0 = i * NBUF
            for k in range(NBUF):
                g = g0 + k
                kr = (k + NBUF - 1) % NBUF
                @pl.when(g + NBUF - 1 < N_GRP)
                def _():
                    issue_loads(t, g + NBUF - 1, kr)
                wait_loads(t, g, k)
                do_scatters(k)
            return carry

        lax.fori_loop(0, N_GRP // NBUF, block, 0)
        plsc.subcore_barrier()
        pltpu.sync_copy(table.at[pl.ds(s * ROWS_PT, ROWS_PT)],
                        out_hbm.at[t, c].at[pl.ds(s * ROWS_PT, ROWS_PT)])
        plsc.subcore_barrier()


@functools.cache
def _get_agg_call():
    scratch = (
        [pltpu.VMEM((128, RNN), jnp.float32) for _ in range(NBUF)]
        + [pltpu.VMEM((128,), jnp.int32) for _ in range(NBUF)]
        + [pltpu.VMEM((128,), jnp.int32) for _ in range(NBUF)]
        + [pltpu.SemaphoreType.DMA for _ in range(NBUF)]
        + [pltpu.SemaphoreType.DMA for _ in range(NBUF)]
        + [pltpu.VMEM_SHARED((NTAB, RNN), jnp.float32)]
    )
    return pl.kernel(
        _agg_body,
        out_type=jax.ShapeDtypeStruct((T, NC, NTAB, RNN), jnp.float32),
        mesh=plsc.VectorSubcoreMesh(core_axis_name="c", subcore_axis_name="s"),
        scratch_types=scratch,
    )


def _agg_run(hs_all, gidx, zeros_tab):
    return _get_agg_call()(hs_all, gidx, zeros_tab)


def _granule_idx(nodes):
    g = jnp.stack([2 * nodes, 2 * nodes + 1], axis=-1)
    return g.reshape(NC * NS, N_GRP, 128)


def _node_body(dte_ref, dn_ref, aggp_ref,
               wenc_t_ref, benc_t_ref, wcat_t_ref, b_t_ref,
               wenc_n_ref, benc_n_ref, wee_ref, bee_ref,
               wcat_n_ref, b_n_ref, wout_ref, bout_ref, out_ref):
    ht = jnp.zeros((TILE_N, RNN), jnp.float32)
    ct = jnp.zeros((TILE_N, RNN), jnp.float32)
    hn = jnp.zeros((TILE_N, RNN), jnp.float32)
    cn = jnp.zeros((TILE_N, RNN), jnp.float32)
    wenc_t = wenc_t_ref[...]
    wenc_n = wenc_n_ref[...]
    wee = wee_ref[...]
    wcat_t = wcat_t_ref[...]
    wcat_n = wcat_n_ref[...]
    for t in range(T):
        xe = dte_ref[t]
        enc_t = jnp.maximum(
            xe[:, 0:1] * wenc_t[0:1, :] + xe[:, 1:2] * wenc_t[1:2, :] + benc_t_ref[...], 0.0)
        ht, ct = _lstm_update(jnp.concatenate([enc_t, ht], axis=1), wcat_t, b_t_ref[...], ct)
        agg = aggp_ref[t, 0] + aggp_ref[t, 1]
        emb_e = jnp.maximum(
            jnp.dot(jnp.concatenate([ht, agg], axis=1), wee,
                    preferred_element_type=jnp.float32) + bee_ref[...], 0.0)
        enc_n = jnp.maximum(dn_ref[t] * wenc_n + benc_n_ref[...], 0.0)
        xh = jnp.concatenate([enc_n, emb_e, hn], axis=1)
        hn, cn = _lstm_update(xh, wcat_n, b_n_ref[...], cn)
        out_ref[t] = jnp.dot(hn, wout_ref[...], preferred_element_type=jnp.float32) + bout_ref[...]


_node_call = pl.pallas_call(
    _node_body,
    grid=(N_PAD // TILE_N,),
    in_specs=[
        pl.BlockSpec((T, TILE_N, 2), lambda i: (0, i, 0)),
        pl.BlockSpec((T, TILE_N, 1), lambda i: (0, i, 0)),
        pl.BlockSpec((T, NC, TILE_N, RNN), lambda i: (0, 0, i, 0)),
        pl.BlockSpec((2, RNN), lambda i: (0, 0)),
        pl.BlockSpec((1, RNN), lambda i: (0, 0)),
        pl.BlockSpec((2 * RNN, 4 * RNN), lambda i: (0, 0)),
        pl.BlockSpec((1, 4 * RNN), lambda i: (0, 0)),
        pl.BlockSpec((1, RNN), lambda i: (0, 0)),
        pl.BlockSpec((1, RNN), lambda i: (0, 0)),
        pl.BlockSpec((2 * RNN, RNN), lambda i: (0, 0)),
        pl.BlockSpec((1, RNN), lambda i: (0, 0)),
        pl.BlockSpec((3 * RNN, 4 * RNN), lambda i: (0, 0)),
        pl.BlockSpec((1, 4 * RNN), lambda i: (0, 0)),
        pl.BlockSpec((RNN, 8), lambda i: (0, 0)),
        pl.BlockSpec((1, 8), lambda i: (0, 0)),
    ],
    out_specs=pl.BlockSpec((T, TILE_N, 8), lambda i: (0, i, 0)),
    out_shape=jax.ShapeDtypeStruct((T, N_PAD, 8), jnp.float32),
)


def kernel(data_nodes, data_temporalEdges, data_spatialEdges, h_nodeRNN, c_nodeRNN,
           h_temporalEdgeRNN, c_temporalEdgeRNN, h_spatialEdgeRNN, c_spatialEdgeRNN,
           edge_index, W_enc_t, b_enc_t, Wih_t, Whh_t, bih_t, bhh_t,
           W_enc_s, b_enc_s, Wih_s, Whh_s, bih_s, bhh_s,
           W_enc_n, b_enc_n, W_ee, b_ee, Wih_n, Whh_n, bih_n, bhh_n,
           W_out, b_out):
    f32 = jnp.float32
    xs = jnp.pad(data_spatialEdges, ((0, 0), (0, E_PAD - E), (0, 0)))
    wcat_s = jnp.concatenate([Wih_s.T, Whh_s.T], axis=0)
    b_s = (bih_s + bhh_s).reshape(1, 4 * RNN)
    hs_all = _spatial_call(xs, W_enc_s.T, b_enc_s.reshape(1, RNN), wcat_s, b_s)

    pad_idx = jnp.full((E_PAD - E,), NTAB - 8, jnp.int32)
    def to_row(n):
        return n + 4 * (n >= HOLE).astype(jnp.int32)
    src = jnp.concatenate([to_row(edge_index[0]), pad_idx])
    dst = jnp.concatenate([to_row(edge_index[1]), pad_idx])
    gidx = jnp.stack([_granule_idx(src), _granule_idx(dst)], axis=2)
    zeros_tab = jnp.zeros((NTAB, RNN), f32)
    aggp = _agg_run(hs_all, gidx, zeros_tab)
    aggp = jnp.concatenate(
        [aggp[:, :, :HOLE], aggp[:, :, HOLE + 4:HOLE + 4 + (N_PAD - HOLE)]], axis=2)

    dte = jnp.pad(data_temporalEdges, ((0, 0), (0, N_PAD - N), (0, 0)))
    dn = jnp.pad(data_nodes, ((0, 0), (0, N_PAD - N), (0, 0)))
    wcat_t = jnp.concatenate([Wih_t.T, Whh_t.T], axis=0)
    b_t = (bih_t + bhh_t).reshape(1, 4 * RNN)
    wcat_n = jnp.concatenate([Wih_n.T, Whh_n.T], axis=0)
    b_n = (bih_n + bhh_n).reshape(1, 4 * RNN)
    wout = jnp.pad(W_out.T, ((0, 0), (0, 8 - 5)))
    bout = jnp.pad(b_out, ((0, 8 - 5))).reshape(1, 8)
    out = _node_call(dte, dn, aggp,
                     W_enc_t.T, b_enc_t.reshape(1, RNN), wcat_t, b_t,
                     W_enc_n.T, b_enc_n.reshape(1, RNN), W_ee.T, b_ee.reshape(1, RNN),
                     wcat_n, b_n, wout, bout)
    return out[:, :N, :5]


# speedup vs baseline: 2.6021x; 1.1237x over previous
import functools

import jax
import jax.numpy as jnp
from jax import lax
from jax.experimental import pallas as pl
from jax.experimental.pallas import tpu as pltpu
from jax.experimental.pallas import tpu_sc as plsc

N = 10000
E = 160000
T = 4
RNN = 64
N_PAD = 10240
NTAB = 11264
HOLE = NTAB // 2
E_PAD = 163840
TILE_E = 2048
TILE_N = 512
NC = 2
NS = 16
PER_W = E_PAD // (NC * NS)
N_CH = PER_W // 128
ROWS_PT = NTAB // NS


def _sig(x):
    return 0.5 * jnp.tanh(0.5 * x) + 0.5


def _lstm_update(xh, wcat, b, c):
    g = jnp.dot(xh, wcat, preferred_element_type=jnp.float32) + b
    i, f, gg, o = jnp.split(g, 4, axis=-1)
    c_new = _sig(f) * c + _sig(i) * jnp.tanh(gg)
    h_new = _sig(o) * jnp.tanh(c_new)
    return h_new, c_new


def _spatial_body(xs_ref, wenc_ref, benc_ref, wcat_ref, b_ref, out_ref):
    h = jnp.zeros((TILE_E, RNN), jnp.float32)
    c = jnp.zeros((TILE_E, RNN), jnp.float32)
    wenc = wenc_ref[...]
    wcat = wcat_ref[...]
    b = b_ref[...]
    benc = benc_ref[...]
    for t in range(T):
        x = xs_ref[t]
        enc = jnp.maximum(x[:, 0:1] * wenc[0:1, :] + x[:, 1:2] * wenc[1:2, :] + benc, 0.0)
        h, c = _lstm_update(jnp.concatenate([enc, h], axis=1), wcat, b, c)
        out_ref[t] = h


_spatial_call = pl.pallas_call(
    _spatial_body,
    grid=(E_PAD // TILE_E,),
    in_specs=[
        pl.BlockSpec((T, TILE_E, 2), lambda i: (0, i, 0)),
        pl.BlockSpec((2, RNN), lambda i: (0, 0)),
        pl.BlockSpec((1, RNN), lambda i: (0, 0)),
        pl.BlockSpec((2 * RNN, 4 * RNN), lambda i: (0, 0)),
        pl.BlockSpec((1, 4 * RNN), lambda i: (0, 0)),
    ],
    out_specs=pl.BlockSpec((T, TILE_E, RNN), lambda i: (0, i, 0)),
    out_shape=jax.ShapeDtypeStruct((T, E_PAD, RNN), jnp.float32),
)


N_GRP = PER_W // 64


NBUF = 2


def _agg_body(hs_hbm, gidx_hbm, zeros_hbm, out_hbm, *scratch):
    bufs = scratch[0:NBUF]
    idx_ss = scratch[NBUF:2 * NBUF]
    idx_ds = scratch[2 * NBUF:3 * NBUF]
    load_sems = scratch[3 * NBUF:4 * NBUF]
    scat_sems = scratch[4 * NBUF:5 * NBUF]
    table = scratch[5 * NBUF]
    c = lax.axis_index("c")
    s = lax.axis_index("s")
    w = c * NS + s
    base = w * PER_W

    def issue_loads(t, g, k):
        pltpu.async_copy(hs_hbm.at[t].at[pl.ds(base + g * 64, 64)],
                         bufs[k].at[pl.ds(0, 64)], load_sems[k])
        pltpu.async_copy(gidx_hbm.at[w, g, 0], idx_ss[k], load_sems[k])
        pltpu.async_copy(gidx_hbm.at[w, g, 1], idx_ds[k], load_sems[k])

    def wait_loads(t, g, k):
        pltpu.make_async_copy(hs_hbm.at[t].at[pl.ds(base + g * 64, 64)],
                              bufs[k].at[pl.ds(0, 64)], load_sems[k]).wait()
        pltpu.make_async_copy(gidx_hbm.at[w, g, 0], idx_ss[k], load_sems[k]).wait()
        pltpu.make_async_copy(gidx_hbm.at[w, g, 1], idx_ds[k], load_sems[k]).wait()

    def do_scatters(k):
        pltpu.async_copy(bufs[k], table.at[idx_ss[k]], scat_sems[k], add=True)
        pltpu.async_copy(bufs[k], table.at[idx_ds[k]], scat_sems[k], add=True)
        pltpu.make_async_copy(bufs[k], table.at[idx_ss[k]], scat_sems[k]).wait()
        pltpu.make_async_copy(bufs[k], table.at[idx_ds[k]], scat_sems[k]).wait()

    for t in range(T):
        pltpu.sync_copy(zeros_hbm.at[pl.ds(s * ROWS_PT, ROWS_PT)],
                        table.at[pl.ds(s * ROWS_PT, ROWS_PT)])
        plsc.subcore_barrier()

        for k in range(NBUF - 1):
            issue_loads(t, k, k)

        def block(i, carry):
            g0 = i * NBUF
            for k in range(NBUF):
                g = g0 + k
                kr = (k + NBUF - 1) % NBUF
                @pl.when(g + NBUF - 1 < N_GRP)
                def _():
                    issue_loads(t, g + NBUF - 1, kr)
                wait_loads(t, g, k)
                do_scatters(k)
            return carry

        lax.fori_loop(0, N_GRP // NBUF, block, 0)
        plsc.subcore_barrier()
        pltpu.sync_copy(table.at[pl.ds(s * ROWS_PT, ROWS_PT)],
                        out_hbm.at[t, c].at[pl.ds(s * ROWS_PT, ROWS_PT)])
        plsc.subcore_barrier()


@functools.cache
def _get_agg_call():
    scratch = (
        [pltpu.VMEM((128, RNN), jnp.float32) for _ in range(NBUF)]
        + [pltpu.VMEM((128,), jnp.int32) for _ in range(NBUF)]
        + [pltpu.VMEM((128,), jnp.int32) for _ in range(NBUF)]
        + [pltpu.SemaphoreType.DMA for _ in range(NBUF)]
        + [pltpu.SemaphoreType.DMA for _ in range(NBUF)]
        + [pltpu.VMEM_SHARED((NTAB, RNN), jnp.float32)]
    )
    return pl.kernel(
        _agg_body,
        out_type=jax.ShapeDtypeStruct((T, NC, NTAB, RNN), jnp.float32),
        mesh=plsc.VectorSubcoreMesh(core_axis_name="c", subcore_axis_name="s"),
        scratch_types=scratch,
    )


def _agg_run(hs_all, gidx, zeros_tab):
    return _get_agg_call()(hs_all, gidx, zeros_tab)


def _granule_idx(nodes):
    g = jnp.stack([2 * nodes, 2 * nodes + 1], axis=-1)
    return g.reshape(NC * NS, N_GRP, 128)


def _node_body(dte_ref, dn_ref, aggp_ref,
               wenc_t_ref, benc_t_ref, wcat_t_ref, b_t_ref,
               wenc_n_ref, benc_n_ref, wee_ref, bee_ref,
               wcat_n_ref, b_n_ref, wout_ref, bout_ref, out_ref):
    ht = jnp.zeros((TILE_N, RNN), jnp.float32)
    ct = jnp.zeros((TILE_N, RNN), jnp.float32)
    hn = jnp.zeros((TILE_N, RNN), jnp.float32)
    cn = jnp.zeros((TILE_N, RNN), jnp.float32)
    wenc_t = wenc_t_ref[...]
    wenc_n = wenc_n_ref[...]
    wee = wee_ref[...]
    wcat_t = wcat_t_ref[...]
    wcat_n = wcat_n_ref[...]
    for t in range(T):
        xe = dte_ref[t]
        enc_t = jnp.maximum(
            xe[:, 0:1] * wenc_t[0:1, :] + xe[:, 1:2] * wenc_t[1:2, :] + benc_t_ref[...], 0.0)
        ht, ct = _lstm_update(jnp.concatenate([enc_t, ht], axis=1), wcat_t, b_t_ref[...], ct)
        agg = aggp_ref[t, 0] + aggp_ref[t, 1]
        emb_e = jnp.maximum(
            jnp.dot(jnp.concatenate([ht, agg], axis=1), wee,
                    preferred_element_type=jnp.float32) + bee_ref[...], 0.0)
        enc_n = jnp.maximum(dn_ref[t] * wenc_n + benc_n_ref[...], 0.0)
        xh = jnp.concatenate([enc_n, emb_e, hn], axis=1)
        hn, cn = _lstm_update(xh, wcat_n, b_n_ref[...], cn)
        out_ref[t] = jnp.dot(hn, wout_ref[...], preferred_element_type=jnp.float32) + bout_ref[...]


_node_call = pl.pallas_call(
    _node_body,
    grid=(N_PAD // TILE_N,),
    in_specs=[
        pl.BlockSpec((T, TILE_N, 2), lambda i: (0, i, 0)),
        pl.BlockSpec((T, TILE_N, 1), lambda i: (0, i, 0)),
        pl.BlockSpec((T, NC, TILE_N, RNN),
                     lambda i: (0, 0, jnp.where(i >= HOLE // TILE_N, i + 1, i), 0)),
        pl.BlockSpec((2, RNN), lambda i: (0, 0)),
        pl.BlockSpec((1, RNN), lambda i: (0, 0)),
        pl.BlockSpec((2 * RNN, 4 * RNN), lambda i: (0, 0)),
        pl.BlockSpec((1, 4 * RNN), lambda i: (0, 0)),
        pl.BlockSpec((1, RNN), lambda i: (0, 0)),
        pl.BlockSpec((1, RNN), lambda i: (0, 0)),
        pl.BlockSpec((2 * RNN, RNN), lambda i: (0, 0)),
        pl.BlockSpec((1, RNN), lambda i: (0, 0)),
        pl.BlockSpec((3 * RNN, 4 * RNN), lambda i: (0, 0)),
        pl.BlockSpec((1, 4 * RNN), lambda i: (0, 0)),
        pl.BlockSpec((RNN, 8), lambda i: (0, 0)),
        pl.BlockSpec((1, 8), lambda i: (0, 0)),
    ],
    out_specs=pl.BlockSpec((T, TILE_N, 8), lambda i: (0, i, 0)),
    out_shape=jax.ShapeDtypeStruct((T, N_PAD, 8), jnp.float32),
)


def kernel(data_nodes, data_temporalEdges, data_spatialEdges, h_nodeRNN, c_nodeRNN,
           h_temporalEdgeRNN, c_temporalEdgeRNN, h_spatialEdgeRNN, c_spatialEdgeRNN,
           edge_index, W_enc_t, b_enc_t, Wih_t, Whh_t, bih_t, bhh_t,
           W_enc_s, b_enc_s, Wih_s, Whh_s, bih_s, bhh_s,
           W_enc_n, b_enc_n, W_ee, b_ee, Wih_n, Whh_n, bih_n, bhh_n,
           W_out, b_out):
    f32 = jnp.float32
    xs = jnp.pad(data_spatialEdges, ((0, 0), (0, E_PAD - E), (0, 0)))
    wcat_s = jnp.concatenate([Wih_s.T, Whh_s.T], axis=0)
    b_s = (bih_s + bhh_s).reshape(1, 4 * RNN)
    hs_all = _spatial_call(xs, W_enc_s.T, b_enc_s.reshape(1, RNN), wcat_s, b_s)

    pad_idx = jnp.full((E_PAD - E,), NTAB - 64, jnp.int32)
    def to_row(n):
        return n + TILE_N * (n >= HOLE).astype(jnp.int32)
    src = jnp.concatenate([to_row(edge_index[0]), pad_idx])
    dst = jnp.concatenate([to_row(edge_index[1]), pad_idx])
    gidx = jnp.stack([_granule_idx(src), _granule_idx(dst)], axis=2)
    zeros_tab = jnp.zeros((NTAB, RNN), f32)
    aggp = _agg_run(hs_all, gidx, zeros_tab)

    dte = jnp.pad(data_temporalEdges, ((0, 0), (0, N_PAD - N), (0, 0)))
    dn = jnp.pad(data_nodes, ((0, 0), (0, N_PAD - N), (0, 0)))
    wcat_t = jnp.concatenate([Wih_t.T, Whh_t.T], axis=0)
    b_t = (bih_t + bhh_t).reshape(1, 4 * RNN)
    wcat_n = jnp.concatenate([Wih_n.T, Whh_n.T], axis=0)
    b_n = (bih_n + bhh_n).reshape(1, 4 * RNN)
    wout = jnp.pad(W_out.T, ((0, 0), (0, 8 - 5)))
    bout = jnp.pad(b_out, ((0, 8 - 5))).reshape(1, 8)
    out = _node_call(dte, dn, aggp,
                     W_enc_t.T, b_enc_t.reshape(1, RNN), wcat_t, b_t,
                     W_enc_n.T, b_enc_n.reshape(1, RNN), W_ee.T, b_ee.reshape(1, RNN),
                     wcat_n, b_n, wout, bout)
    return out[:, :N, :5]
